# baseline (speedup 1.0000x reference)
import functools

import jax
import jax.numpy as jnp
from jax import lax
from jax.experimental import pallas as pl
from jax.experimental.pallas import tpu as pltpu
from jax.experimental.pallas import tpu_sc as plsc

B, S, D, ROWS = 4, 8192, 1024, 130
N = B * S
NC, NS, L = 2, 16, 16
NW = NC * NS
NG = NW // 2
DH = D // 2
CH = 32

NSC = 8192
TPG = NSC // NG
NCH = TPG // CH

RP = 256
TB = 512
NTCB = (N - NSC) // TB
OFF = NSC // TB

_mesh = plsc.VectorSubcoreMesh(core_axis_name="c", subcore_axis_name="s")


@functools.partial(
    pl.kernel,
    out_type=jax.ShapeDtypeStruct((NSC, D), jnp.float32),
    mesh=_mesh,
    scratch_types=[
        pltpu.VMEM((ROWS, DH), jnp.float32),
        pltpu.VMEM((CH, DH), jnp.float32),
        pltpu.VMEM((CH, DH), jnp.float32),
        pltpu.VMEM((CH, L), jnp.int32),
        pltpu.VMEM((CH, L), jnp.int32),
        pltpu.SemaphoreType.DMA,
        pltpu.SemaphoreType.DMA,
        pltpu.SemaphoreType.DMA,
        pltpu.SemaphoreType.DMA,
        pltpu.SemaphoreType.DMA,
        pltpu.SemaphoreType.DMA,
    ],
)
def _sc_embed_add(x_hbm, idsb_hbm, table_hbm, out_hbm, tbl,
                  xb0, xb1, ib0, ib1, sx0, sx1, si0, si1, so0, so1):
    wid = lax.axis_index("s") * NC + lax.axis_index("c")
    g = wid // 2
    h = wid % 2
    base = g * TPG
    cbase = h * DH

    pltpu.sync_copy(table_hbm.at[h], tbl)

    xbs, ibs = (xb0, xb1), (ib0, ib1)
    sxs, sis, sos = (sx0, sx1), (si0, si1), (so0, so1)

    def issue(k, b):
        pltpu.async_copy(
            x_hbm.at[pl.ds(base + k * CH, CH), pl.ds(cbase, DH)],
            xbs[b], sxs[b])
        pltpu.async_copy(
            idsb_hbm.at[pl.ds(base + k * CH, CH)], ibs[b], sis[b])

    def wait_in(k, b):
        pltpu.make_async_copy(
            x_hbm.at[pl.ds(base + k * CH, CH), pl.ds(cbase, DH)],
            xbs[b], sxs[b]).wait()
        pltpu.make_async_copy(
            idsb_hbm.at[pl.ds(base + k * CH, CH)], ibs[b], sis[b]).wait()

    def store(k, b):
        pltpu.async_copy(
            xbs[b], out_hbm.at[pl.ds(base + k * CH, CH), pl.ds(cbase, DH)],
            sos[b])

    def wait_store(k, b):
        pltpu.make_async_copy(
            xbs[b], out_hbm.at[pl.ds(base + k * CH, CH), pl.ds(cbase, DH)],
            sos[b]).wait()

    def compute(b):
        xb, ib = xbs[b], ibs[b]

        @plsc.parallel_loop(0, CH, step=1, unroll=4)
        def tok_body(t):
            rid = ib[t][0]
            for c in range(DH // L):
                sl = pl.ds(c * L, L)
                plsc.addupdate(xb.at[t, sl], tbl[rid, sl])

    issue(0, 0)

    def body(j, carry):
        for hh in range(2):
            k = 2 * j + hh
            kp = k + 1
            b, bp = hh, 1 - hh

            @pl.when(kp < NCH)
            def _():
                @pl.when(kp >= 2)
                def _():
                    wait_store(kp - 2, bp)
                issue(kp, bp)

            wait_in(k, b)
            compute(b)
            store(k, b)
        return carry

    lax.fori_loop(0, NCH // 2, body, 0)
    wait_store(NCH - 2, 0)
    wait_store(NCH - 1, 1)


def _tc_body(ids_ref, x_ref, thi_ref, tlo_ref, out_ref):
    idsv = ids_ref[0, 0, :]
    iot = lax.broadcasted_iota(jnp.int32, (TB, RP), 1)
    oh = (idsv[:, None] == iot).astype(jnp.bfloat16)
    acc = jnp.dot(oh, thi_ref[...], preferred_element_type=jnp.float32)
    acc = acc + jnp.dot(oh, tlo_ref[...], preferred_element_type=jnp.float32)
    out_ref[...] = x_ref[...] + acc


_tc_call = pl.pallas_call(
    _tc_body,
    grid=(NTCB,),
    in_specs=[
        pl.BlockSpec((1, 1, TB), lambda i: (i + OFF, 0, 0)),
        pl.BlockSpec((TB, D), lambda i: (i + OFF, 0)),
        pl.BlockSpec((RP, D), lambda i: (0, 0)),
        pl.BlockSpec((RP, D), lambda i: (0, 0)),
    ],
    out_specs=pl.BlockSpec((TB, D), lambda i: (i + OFF, 0)),
    out_shape=jax.ShapeDtypeStruct((N, D), jnp.float32),
)


def kernel(x, instrument_ids, table):
    ids = instrument_ids.reshape(-1).astype(jnp.int32)
    xf = x.reshape(N, D)
    ids_b = jnp.broadcast_to(ids[:, None], (N, L))
    tab2 = table.reshape(ROWS, 2, DH).transpose(1, 0, 2)
    part = _sc_embed_add(xf, ids_b, tab2)
    ids3 = ids.reshape(N // TB, 1, TB)
    thi = jnp.zeros((RP, D), jnp.bfloat16).at[:ROWS].set(
        table.astype(jnp.bfloat16))
    tlo = jnp.zeros((RP, D), jnp.bfloat16).at[:ROWS].set(
        (table - thi[:ROWS].astype(jnp.float32)).astype(jnp.bfloat16))
    tc_full = _tc_call(ids3, xf, thi, tlo)
    out = lax.dynamic_update_slice(tc_full, part, (0, 0))
    return out.reshape(B, S, D)


# speedup vs baseline: 1.0678x; 1.0678x over previous
import functools

import jax
import jax.numpy as jnp
from jax import lax
from jax.experimental import pallas as pl
from jax.experimental.pallas import tpu as pltpu
from jax.experimental.pallas import tpu_sc as plsc

B, S, D, ROWS = 4, 8192, 1024, 130
N = B * S
NC, NS, L = 2, 16, 16
NW = NC * NS
NG = NW // 2
DH = D // 2
CH = 32

NSC = 2048
TPG = NSC // NG
NCH = TPG // CH

RP = 256
TB = 512
NTCB = (N - NSC) // TB
OFF = NSC // TB

_mesh = plsc.VectorSubcoreMesh(core_axis_name="c", subcore_axis_name="s")


@functools.partial(
    pl.kernel,
    out_type=jax.ShapeDtypeStruct((N, D), jnp.float32),
    mesh=_mesh,
    scratch_types=[
        pltpu.VMEM((ROWS, DH), jnp.float32),
        pltpu.VMEM((CH, DH), jnp.float32),
        pltpu.VMEM((CH, DH), jnp.float32),
        pltpu.VMEM((CH, L), jnp.int32),
        pltpu.VMEM((CH, L), jnp.int32),
        pltpu.SemaphoreType.DMA,
        pltpu.SemaphoreType.DMA,
        pltpu.SemaphoreType.DMA,
        pltpu.SemaphoreType.DMA,
        pltpu.SemaphoreType.DMA,
        pltpu.SemaphoreType.DMA,
    ],
)
def _sc_embed_add(x_hbm, idsb_hbm, table_hbm, out_hbm, tbl,
                  xb0, xb1, ib0, ib1, sx0, sx1, si0, si1, so0, so1):
    wid = lax.axis_index("s") * NC + lax.axis_index("c")
    g = wid // 2
    h = wid % 2
    base = g * TPG
    cbase = h * DH

    pltpu.sync_copy(table_hbm.at[h], tbl)

    xbs, ibs = (xb0, xb1), (ib0, ib1)
    sxs, sis, sos = (sx0, sx1), (si0, si1), (so0, so1)

    def issue(k, b):
        pltpu.async_copy(
            x_hbm.at[pl.ds(base + k * CH, CH), pl.ds(cbase, DH)],
            xbs[b], sxs[b])
        pltpu.async_copy(
            idsb_hbm.at[pl.ds(base + k * CH, CH)], ibs[b], sis[b])

    def wait_in(k, b):
        pltpu.make_async_copy(
            x_hbm.at[pl.ds(base + k * CH, CH), pl.ds(cbase, DH)],
            xbs[b], sxs[b]).wait()
        pltpu.make_async_copy(
            idsb_hbm.at[pl.ds(base + k * CH, CH)], ibs[b], sis[b]).wait()

    def store(k, b):
        pltpu.async_copy(
            xbs[b], out_hbm.at[pl.ds(base + k * CH, CH), pl.ds(cbase, DH)],
            sos[b])

    def wait_store(k, b):
        pltpu.make_async_copy(
            xbs[b], out_hbm.at[pl.ds(base + k * CH, CH), pl.ds(cbase, DH)],
            sos[b]).wait()

    def compute(b):
        xb, ib = xbs[b], ibs[b]

        @plsc.parallel_loop(0, CH, step=1, unroll=2)
        def tok_body(t):
            rid = ib[t][0]
            for c in range(DH // L):
                sl = pl.ds(c * L, L)
                plsc.addupdate(xb.at[t, sl], tbl[rid, sl])

    issue(0, 0)

    def body(j, carry):
        for hh in range(2):
            k = 2 * j + hh
            kp = k + 1
            b, bp = hh, 1 - hh

            @pl.when(kp < NCH)
            def _():
                @pl.when(kp >= 2)
                def _():
                    wait_store(kp - 2, bp)
                issue(kp, bp)

            wait_in(k, b)
            compute(b)
            store(k, b)
        return carry

    lax.fori_loop(0, NCH // 2, body, 0)
    wait_store(NCH - 2, 0)
    wait_store(NCH - 1, 1)


def _tc_body(prev_ref, ids_ref, x_ref, thi_ref, tlo_ref, out_ref):
    del prev_ref
    idsv = ids_ref[0, 0, :]
    iot = lax.broadcasted_iota(jnp.int32, (TB, RP), 1)
    oh = (idsv[:, None] == iot).astype(jnp.bfloat16)
    acc = jnp.dot(oh, thi_ref[...], preferred_element_type=jnp.float32)
    acc = acc + jnp.dot(oh, tlo_ref[...], preferred_element_type=jnp.float32)
    out_ref[...] = x_ref[...] + acc


_tc_call = pl.pallas_call(
    _tc_body,
    grid=(NTCB,),
    in_specs=[
        pl.BlockSpec(memory_space=pl.ANY),
        pl.BlockSpec((1, 1, TB), lambda i: (i + OFF, 0, 0)),
        pl.BlockSpec((TB, D), lambda i: (i + OFF, 0)),
        pl.BlockSpec((RP, D), lambda i: (0, 0)),
        pl.BlockSpec((RP, D), lambda i: (0, 0)),
    ],
    out_specs=pl.BlockSpec((TB, D), lambda i: (i + OFF, 0)),
    out_shape=jax.ShapeDtypeStruct((N, D), jnp.float32),
    input_output_aliases={0: 0},
)


def kernel(x, instrument_ids, table):
    ids = instrument_ids.reshape(-1).astype(jnp.int32)
    xf = x.reshape(N, D)
    ids_b = jnp.broadcast_to(ids[:, None], (N, L))
    tab2 = table.reshape(ROWS, 2, DH).transpose(1, 0, 2)
    part = _sc_embed_add(xf, ids_b, tab2)
    ids3 = ids.reshape(N // TB, 1, TB)
    thi = jnp.zeros((RP, D), jnp.bfloat16).at[:ROWS].set(
        table.astype(jnp.bfloat16))
    tlo = jnp.zeros((RP, D), jnp.bfloat16).at[:ROWS].set(
        (table - thi[:ROWS].astype(jnp.float32)).astype(jnp.bfloat16))
    out = _tc_call(part, ids3, xf, thi, tlo)
    return out.reshape(B, S, D)


# speedup vs baseline: 1.0750x; 1.0067x over previous
import functools

import jax
import jax.numpy as jnp
from jax import lax
from jax.experimental import pallas as pl
from jax.experimental.pallas import tpu as pltpu
from jax.experimental.pallas import tpu_sc as plsc

B, S, D, ROWS = 4, 8192, 1024, 130
N = B * S
NC, NS, L = 2, 16, 16
NW = NC * NS
NG = NW // 2
DH = D // 2
CH = 32

NSC = 1024
TPG = NSC // NG
NCH = TPG // CH

RP = 256
TB = 512
NTCB = (N - NSC) // TB
OFF = NSC // TB

_mesh = plsc.VectorSubcoreMesh(core_axis_name="c", subcore_axis_name="s")


@functools.partial(
    pl.kernel,
    out_type=jax.ShapeDtypeStruct((N, D), jnp.float32),
    mesh=_mesh,
    scratch_types=[
        pltpu.VMEM((ROWS, DH), jnp.float32),
        pltpu.VMEM((CH, DH), jnp.float32),
        pltpu.VMEM((CH, DH), jnp.float32),
        pltpu.VMEM((CH, L), jnp.int32),
        pltpu.VMEM((CH, L), jnp.int32),
        pltpu.SemaphoreType.DMA,
        pltpu.SemaphoreType.DMA,
        pltpu.SemaphoreType.DMA,
        pltpu.SemaphoreType.DMA,
        pltpu.SemaphoreType.DMA,
        pltpu.SemaphoreType.DMA,
    ],
)
def _sc_embed_add(x_hbm, idsb_hbm, table_hbm, out_hbm, tbl,
                  xb0, xb1, ib0, ib1, sx0, sx1, si0, si1, so0, so1):
    wid = lax.axis_index("s") * NC + lax.axis_index("c")
    g = wid // 2
    h = wid % 2
    base = g * TPG
    cbase = h * DH

    pltpu.sync_copy(table_hbm.at[h], tbl)

    xbs, ibs = (xb0, xb1), (ib0, ib1)
    sxs, sis, sos = (sx0, sx1), (si0, si1), (so0, so1)

    def issue(k, b):
        pltpu.async_copy(
            x_hbm.at[pl.ds(base + k * CH, CH), pl.ds(cbase, DH)],
            xbs[b], sxs[b])
        pltpu.async_copy(
            idsb_hbm.at[pl.ds(base + k * CH, CH)], ibs[b], sis[b])

    def wait_in(k, b):
        pltpu.make_async_copy(
            x_hbm.at[pl.ds(base + k * CH, CH), pl.ds(cbase, DH)],
            xbs[b], sxs[b]).wait()
        pltpu.make_async_copy(
            idsb_hbm.at[pl.ds(base + k * CH, CH)], ibs[b], sis[b]).wait()

    def store(k, b):
        pltpu.async_copy(
            xbs[b], out_hbm.at[pl.ds(base + k * CH, CH), pl.ds(cbase, DH)],
            sos[b])

    def wait_store(k, b):
        pltpu.make_async_copy(
            xbs[b], out_hbm.at[pl.ds(base + k * CH, CH), pl.ds(cbase, DH)],
            sos[b]).wait()

    def compute(b):
        xb, ib = xbs[b], ibs[b]

        @plsc.parallel_loop(0, CH, step=1, unroll=2)
        def tok_body(t):
            rid = ib[t][0]
            for c in range(DH // L):
                sl = pl.ds(c * L, L)
                plsc.addupdate(xb.at[t, sl], tbl[rid, sl])

    issue(0, 0)

    def body(j, carry):
        for hh in range(2):
            k = 2 * j + hh
            kp = k + 1
            b, bp = hh, 1 - hh

            @pl.when(kp < NCH)
            def _():
                @pl.when(kp >= 2)
                def _():
                    wait_store(kp - 2, bp)
                issue(kp, bp)

            wait_in(k, b)
            compute(b)
            store(k, b)
        return carry

    lax.fori_loop(0, NCH // 2, body, 0)
    wait_store(NCH - 2, 0)
    wait_store(NCH - 1, 1)


def _tc_body(prev_ref, ids_ref, x_ref, thi_ref, tlo_ref, out_ref):
    del prev_ref
    idsv = ids_ref[0, 0, :]
    iot = lax.broadcasted_iota(jnp.int32, (TB, RP), 1)
    oh = (idsv[:, None] == iot).astype(jnp.bfloat16)
    acc = jnp.dot(oh, thi_ref[...], preferred_element_type=jnp.float32)
    acc = acc + jnp.dot(oh, tlo_ref[...], preferred_element_type=jnp.float32)
    out_ref[...] = x_ref[...] + acc


_tc_call = pl.pallas_call(
    _tc_body,
    grid=(NTCB,),
    in_specs=[
        pl.BlockSpec(memory_space=pl.ANY),
        pl.BlockSpec((1, 1, TB), lambda i: (i + OFF, 0, 0)),
        pl.BlockSpec((TB, D), lambda i: (i + OFF, 0)),
        pl.BlockSpec((RP, D), lambda i: (0, 0)),
        pl.BlockSpec((RP, D), lambda i: (0, 0)),
    ],
    out_specs=pl.BlockSpec((TB, D), lambda i: (i + OFF, 0)),
    out_shape=jax.ShapeDtypeStruct((N, D), jnp.float32),
    input_output_aliases={0: 0},
)


def kernel(x, instrument_ids, table):
    ids = instrument_ids.reshape(-1).astype(jnp.int32)
    xf = x.reshape(N, D)
    ids_b = jnp.broadcast_to(ids[:, None], (N, L))
    tab2 = table.reshape(ROWS, 2, DH).transpose(1, 0, 2)
    part = _sc_embed_add(xf, ids_b, tab2)
    ids3 = ids.reshape(N // TB, 1, TB)
    thi = jnp.zeros((RP, D), jnp.bfloat16).at[:ROWS].set(
        table.astype(jnp.bfloat16))
    tlo = jnp.zeros((RP, D), jnp.bfloat16).at[:ROWS].set(
        (table - thi[:ROWS].astype(jnp.float32)).astype(jnp.bfloat16))
    out = _tc_call(part, ids3, xf, thi, tlo)
    return out.reshape(B, S, D)


# speedup vs baseline: 1.2133x; 1.1286x over previous
import functools

import jax
import jax.numpy as jnp
from jax import lax
from jax.experimental import pallas as pl
from jax.experimental.pallas import tpu as pltpu
from jax.experimental.pallas import tpu_sc as plsc

B, S, D, ROWS = 4, 8192, 1024, 130
N = B * S
NC, NS, L = 2, 16, 16
NW = NC * NS
NG = NW // 2
DH = D // 2
CH = 32

NSC = 1024
TPG = NSC // NG
NCH = TPG // CH

RP = 256
TB = 1024
NTCB = (N - NSC) // TB
OFF = NSC // TB

_mesh = plsc.VectorSubcoreMesh(core_axis_name="c", subcore_axis_name="s")


@functools.partial(
    pl.kernel,
    out_type=jax.ShapeDtypeStruct((N, D), jnp.float32),
    mesh=_mesh,
    scratch_types=[
        pltpu.VMEM((ROWS, DH), jnp.float32),
        pltpu.VMEM((CH, DH), jnp.float32),
        pltpu.VMEM((CH, DH), jnp.float32),
        pltpu.VMEM((CH, L), jnp.int32),
        pltpu.VMEM((CH, L), jnp.int32),
        pltpu.SemaphoreType.DMA,
        pltpu.SemaphoreType.DMA,
        pltpu.SemaphoreType.DMA,
        pltpu.SemaphoreType.DMA,
        pltpu.SemaphoreType.DMA,
        pltpu.SemaphoreType.DMA,
    ],
)
def _sc_embed_add(x_hbm, idsb_hbm, table_hbm, out_hbm, tbl,
                  xb0, xb1, ib0, ib1, sx0, sx1, si0, si1, so0, so1):
    wid = lax.axis_index("s") * NC + lax.axis_index("c")
    g = wid // 2
    h = wid % 2
    base = g * TPG
    cbase = h * DH

    pltpu.sync_copy(table_hbm.at[h], tbl)

    xbs, ibs = (xb0, xb1), (ib0, ib1)
    sxs, sis, sos = (sx0, sx1), (si0, si1), (so0, so1)

    def issue(k, b):
        pltpu.async_copy(
            x_hbm.at[pl.ds(base + k * CH, CH), pl.ds(cbase, DH)],
            xbs[b], sxs[b])
        pltpu.async_copy(
            idsb_hbm.at[pl.ds(base + k * CH, CH)], ibs[b], sis[b])

    def wait_in(k, b):
        pltpu.make_async_copy(
            x_hbm.at[pl.ds(base + k * CH, CH), pl.ds(cbase, DH)],
            xbs[b], sxs[b]).wait()
        pltpu.make_async_copy(
            idsb_hbm.at[pl.ds(base + k * CH, CH)], ibs[b], sis[b]).wait()

    def store(k, b):
        pltpu.async_copy(
            xbs[b], out_hbm.at[pl.ds(base + k * CH, CH), pl.ds(cbase, DH)],
            sos[b])

    def wait_store(k, b):
        pltpu.make_async_copy(
            xbs[b], out_hbm.at[pl.ds(base + k * CH, CH), pl.ds(cbase, DH)],
            sos[b]).wait()

    def compute(b):
        xb, ib = xbs[b], ibs[b]

        @plsc.parallel_loop(0, CH, step=1, unroll=2)
        def tok_body(t):
            rid = ib[t][0]
            for c in range(DH // L):
                sl = pl.ds(c * L, L)
                plsc.addupdate(xb.at[t, sl], tbl[rid, sl])

    issue(0, 0)

    def body(j, carry):
        for hh in range(2):
            k = 2 * j + hh
            kp = k + 1
            b, bp = hh, 1 - hh

            @pl.when(kp < NCH)
            def _():
                @pl.when(kp >= 2)
                def _():
                    wait_store(kp - 2, bp)
                issue(kp, bp)

            wait_in(k, b)
            compute(b)
            store(k, b)
        return carry

    lax.fori_loop(0, NCH // 2, body, 0)
    wait_store(NCH - 2, 0)
    wait_store(NCH - 1, 1)


def _tc_body(prev_ref, ids_ref, x_ref, thi_ref, tlo_ref, out_ref):
    del prev_ref
    idsv = ids_ref[0, 0, :]
    iot = lax.broadcasted_iota(jnp.int32, (TB, RP), 1)
    oh = (idsv[:, None] == iot).astype(jnp.bfloat16)
    acc = jnp.dot(oh, thi_ref[...], preferred_element_type=jnp.float32)
    acc = acc + jnp.dot(oh, tlo_ref[...], preferred_element_type=jnp.float32)
    out_ref[...] = x_ref[...] + acc


_tc_call = pl.pallas_call(
    _tc_body,
    grid=(NTCB,),
    in_specs=[
        pl.BlockSpec(memory_space=pl.ANY),
        pl.BlockSpec((1, 1, TB), lambda i: (i + OFF, 0, 0)),
        pl.BlockSpec((TB, D), lambda i: (i + OFF, 0)),
        pl.BlockSpec((RP, D), lambda i: (0, 0)),
        pl.BlockSpec((RP, D), lambda i: (0, 0)),
    ],
    out_specs=pl.BlockSpec((TB, D), lambda i: (i + OFF, 0)),
    out_shape=jax.ShapeDtypeStruct((N, D), jnp.float32),
    input_output_aliases={0: 0},
)


def kernel(x, instrument_ids, table):
    ids = instrument_ids.reshape(-1).astype(jnp.int32)
    xf = x.reshape(N, D)
    ids_b = jnp.broadcast_to(ids[:, None], (N, L))
    tab2 = table.reshape(ROWS, 2, DH).transpose(1, 0, 2)
    part = _sc_embed_add(xf, ids_b, tab2)
    ids3 = ids.reshape(N // TB, 1, TB)
    thi = jnp.zeros((RP, D), jnp.bfloat16).at[:ROWS].set(
        table.astype(jnp.bfloat16))
    tlo = jnp.zeros((RP, D), jnp.bfloat16).at[:ROWS].set(
        (table - thi[:ROWS].astype(jnp.float32)).astype(jnp.bfloat16))
    out = _tc_call(part, ids3, xf, thi, tlo)
    return out.reshape(B, S, D)


# speedup vs baseline: 1.2633x; 1.0413x over previous
import functools

import jax
import jax.numpy as jnp
from jax import lax
from jax.experimental import pallas as pl
from jax.experimental.pallas import tpu as pltpu
from jax.experimental.pallas import tpu_sc as plsc

B, S, D, ROWS = 4, 8192, 1024, 130
N = B * S
NC, NS, L = 2, 16, 16
NW = NC * NS
NG = NW // 2
DH = D // 2
CH = 32

NSC = 1024
TPG = NSC // NG
NCH = TPG // CH

RP = 256
TB = 2048
NTCB = (N - NSC) // TB
OFF = NSC // TB

_mesh = plsc.VectorSubcoreMesh(core_axis_name="c", subcore_axis_name="s")


@functools.partial(
    pl.kernel,
    out_type=jax.ShapeDtypeStruct((N, D), jnp.float32),
    mesh=_mesh,
    scratch_types=[
        pltpu.VMEM((ROWS, DH), jnp.float32),
        pltpu.VMEM((CH, DH), jnp.float32),
        pltpu.VMEM((CH, DH), jnp.float32),
        pltpu.VMEM((CH, L), jnp.int32),
        pltpu.VMEM((CH, L), jnp.int32),
        pltpu.SemaphoreType.DMA,
        pltpu.SemaphoreType.DMA,
        pltpu.SemaphoreType.DMA,
        pltpu.SemaphoreType.DMA,
        pltpu.SemaphoreType.DMA,
        pltpu.SemaphoreType.DMA,
    ],
)
def _sc_embed_add(x_hbm, idsb_hbm, table_hbm, out_hbm, tbl,
                  xb0, xb1, ib0, ib1, sx0, sx1, si0, si1, so0, so1):
    wid = lax.axis_index("s") * NC + lax.axis_index("c")
    g = wid // 2
    h = wid % 2
    base = g * TPG
    cbase = h * DH

    pltpu.sync_copy(table_hbm.at[h], tbl)

    xbs, ibs = (xb0, xb1), (ib0, ib1)
    sxs, sis, sos = (sx0, sx1), (si0, si1), (so0, so1)

    def issue(k, b):
        pltpu.async_copy(
            x_hbm.at[pl.ds(base + k * CH, CH), pl.ds(cbase, DH)],
            xbs[b], sxs[b])
        pltpu.async_copy(
            idsb_hbm.at[pl.ds(base + k * CH, CH)], ibs[b], sis[b])

    def wait_in(k, b):
        pltpu.make_async_copy(
            x_hbm.at[pl.ds(base + k * CH, CH), pl.ds(cbase, DH)],
            xbs[b], sxs[b]).wait()
        pltpu.make_async_copy(
            idsb_hbm.at[pl.ds(base + k * CH, CH)], ibs[b], sis[b]).wait()

    def store(k, b):
        pltpu.async_copy(
            xbs[b], out_hbm.at[pl.ds(base + k * CH, CH), pl.ds(cbase, DH)],
            sos[b])

    def wait_store(k, b):
        pltpu.make_async_copy(
            xbs[b], out_hbm.at[pl.ds(base + k * CH, CH), pl.ds(cbase, DH)],
            sos[b]).wait()

    def compute(b):
        xb, ib = xbs[b], ibs[b]

        @plsc.parallel_loop(0, CH, step=1, unroll=2)
        def tok_body(t):
            rid = ib[t][0]
            for c in range(DH // L):
                sl = pl.ds(c * L, L)
                plsc.addupdate(xb.at[t, sl], tbl[rid, sl])

    issue(0, 0)

    def body(j, carry):
        for hh in range(2):
            k = 2 * j + hh
            kp = k + 1
            b, bp = hh, 1 - hh

            @pl.when(kp < NCH)
            def _():
                @pl.when(kp >= 2)
                def _():
                    wait_store(kp - 2, bp)
                issue(kp, bp)

            wait_in(k, b)
            compute(b)
            store(k, b)
        return carry

    lax.fori_loop(0, NCH // 2, body, 0)
    wait_store(NCH - 2, 0)
    wait_store(NCH - 1, 1)


def _tc_body(prev_ref, ids_ref, x_ref, thi_ref, tlo_ref, out_ref):
    del prev_ref
    idsv = ids_ref[0, 0, :]
    iot = lax.broadcasted_iota(jnp.int32, (TB, RP), 1)
    oh = (idsv[:, None] == iot).astype(jnp.bfloat16)
    acc = jnp.dot(oh, thi_ref[...], preferred_element_type=jnp.float32)
    acc = acc + jnp.dot(oh, tlo_ref[...], preferred_element_type=jnp.float32)
    out_ref[...] = x_ref[...] + acc


_tc_call = pl.pallas_call(
    _tc_body,
    grid=(NTCB,),
    in_specs=[
        pl.BlockSpec(memory_space=pl.ANY),
        pl.BlockSpec((1, 1, TB), lambda i: (i + OFF, 0, 0)),
        pl.BlockSpec((TB, D), lambda i: (i + OFF, 0)),
        pl.BlockSpec((RP, D), lambda i: (0, 0)),
        pl.BlockSpec((RP, D), lambda i: (0, 0)),
    ],
    out_specs=pl.BlockSpec((TB, D), lambda i: (i + OFF, 0)),
    out_shape=jax.ShapeDtypeStruct((N, D), jnp.float32),
    input_output_aliases={0: 0},
)


def kernel(x, instrument_ids, table):
    ids = instrument_ids.reshape(-1).astype(jnp.int32)
    xf = x.reshape(N, D)
    ids_b = jnp.broadcast_to(ids[:, None], (N, L))
    tab2 = table.reshape(ROWS, 2, DH).transpose(1, 0, 2)
    part = _sc_embed_add(xf, ids_b, tab2)
    ids3 = ids.reshape(N // TB, 1, TB)
    thi = jnp.zeros((RP, D), jnp.bfloat16).at[:ROWS].set(
        table.astype(jnp.bfloat16))
    tlo = jnp.zeros((RP, D), jnp.bfloat16).at[:ROWS].set(
        (table - thi[:ROWS].astype(jnp.float32)).astype(jnp.bfloat16))
    out = _tc_call(part, ids3, xf, thi, tlo)
    return out.reshape(B, S, D)
